# baseline (speedup 1.0000x reference)
import functools

import jax
import jax.numpy as jnp
import numpy as np
from jax import lax
from jax.experimental import pallas as pl
from jax.experimental.pallas import tpu as pltpu
from jax.experimental.pallas import tpu_sc as plsc

_HIDDEN = 1024
_VOCAB = 100000
_NUM_CENTROIDS = 1000
_TOP_K = 8
_PER_CENTROID = _VOCAB // _NUM_CENTROIDS
_NUM_TOKENS = 128
_FMIN = float(np.finfo(np.float32).min)

_SC_NC, _SC_NS = 2, 16
_SC_ACTIVE = 25
_SC_CHUNKS = _NUM_CENTROIDS // _SC_ACTIVE


_SC_PER_TILE = 3128
_VOCAB_PAD = _SC_PER_TILE * _SC_NC * _SC_NS
_SC_SLICE = _VOCAB_PAD // _SC_NS


def _sc_invperm(ord1d, vals1d, zeros1d):
    mesh = plsc.VectorSubcoreMesh(core_axis_name="c", subcore_axis_name="s")

    @functools.partial(
        pl.kernel,
        out_type=jax.ShapeDtypeStruct((_SC_NC * _VOCAB_PAD,), jnp.int32),
        mesh=mesh,
        scratch_types=[
            pltpu.VMEM((_SC_PER_TILE,), jnp.int32),
            pltpu.VMEM((_SC_PER_TILE,), jnp.int32),
            pltpu.VMEM((_SC_SLICE,), jnp.int32),
            pltpu.VMEM_SHARED((_VOCAB_PAD,), jnp.int32),
            pltpu.SemaphoreType.DMA,
        ],
    )
    def k(ord_hbm, vals_hbm, zeros_hbm, out_hbm, idx_v, val_v, zv, buf, sem):
        cid = lax.axis_index("c")
        sid = lax.axis_index("s")
        wid = sid * _SC_NC + cid
        pltpu.sync_copy(zeros_hbm, zv)
        pltpu.sync_copy(zv, buf.at[pl.ds(sid * _SC_SLICE, _SC_SLICE)])
        base = wid * _SC_PER_TILE
        pltpu.sync_copy(ord_hbm.at[pl.ds(base, _SC_PER_TILE)], idx_v)
        pltpu.sync_copy(vals_hbm.at[pl.ds(base, _SC_PER_TILE)], val_v)
        plsc.subcore_barrier()
        pltpu.async_copy(val_v, buf.at[idx_v], sem).wait()
        plsc.subcore_barrier()
        pltpu.sync_copy(buf.at[pl.ds(sid * _SC_SLICE, _SC_SLICE)], zv)
        pltpu.sync_copy(
            zv, out_hbm.at[pl.ds(cid * _VOCAB_PAD + sid * _SC_SLICE, _SC_SLICE)])

    return k(ord1d, vals1d, zeros1d)


_BLK_V = 4096


def _main_body(h_ref, wc_ref, cid_ref, w_ref, out_ref, top_s):
    @pl.when(pl.program_id(0) == 0)
    def _():
        scores = lax.dot_general(
            h_ref[...], wc_ref[...], (((1,), (1,)), ((), ())),
            preferred_element_type=jnp.float32)
        iota = lax.broadcasted_iota(jnp.int32, scores.shape, 1)
        s = scores
        cols = []
        for _ in range(_TOP_K):
            m = jnp.max(s, axis=1, keepdims=True)
            idx = jnp.min(jnp.where(s >= m, iota, jnp.int32(2**30)),
                          axis=1, keepdims=True)
            cols.append(idx)
            s = jnp.where(iota == idx, _FMIN, s)
        top_s[...] = jnp.concatenate(cols, axis=1)

    h = h_ref[...].astype(jnp.bfloat16)
    w = w_ref[...].astype(jnp.bfloat16)
    logits = lax.dot_general(
        h, w, (((1,), (1,)), ((), ())),
        preferred_element_type=jnp.float32)
    cid2 = cid_ref[...]
    cid = cid2[0:1] + cid2[1:2]
    top = top_s[...]
    sel = cid == top[:, 0:1]
    for kk in range(1, _TOP_K):
        sel = sel | (cid == top[:, kk:kk + 1])
    out_ref[...] = jnp.where(sel, logits, jnp.float32(_FMIN))


def kernel(hidden_states, lm_head_weight, W_c, token_ordering):
    ord1d = jnp.concatenate([
        token_ordering,
        jnp.arange(_VOCAB, _VOCAB_PAD, dtype=jnp.int32)])
    vals1d = jnp.concatenate([
        jnp.arange(_VOCAB, dtype=jnp.int32) // _PER_CENTROID,
        jnp.zeros(_VOCAB_PAD - _VOCAB, jnp.int32)])
    zeros1d = jnp.zeros(_SC_SLICE, jnp.int32)

    cid2d = _sc_invperm(ord1d, vals1d, zeros1d).reshape(_SC_NC, _VOCAB_PAD)

    n_blocks = -(-_VOCAB // _BLK_V)
    out = pl.pallas_call(
        _main_body,
        grid=(n_blocks,),
        in_specs=[
            pl.BlockSpec((_NUM_TOKENS, _HIDDEN), lambda v: (0, 0)),
            pl.BlockSpec((_NUM_CENTROIDS, _HIDDEN), lambda v: (0, 0)),
            pl.BlockSpec((_SC_NC, _BLK_V), lambda v: (0, v)),
            pl.BlockSpec((_BLK_V, _HIDDEN), lambda v: (v, 0)),
        ],
        out_specs=pl.BlockSpec((_NUM_TOKENS, _BLK_V), lambda v: (0, v)),
        out_shape=jax.ShapeDtypeStruct((_NUM_TOKENS, _VOCAB), jnp.float32),
        scratch_shapes=[pltpu.VMEM((_NUM_TOKENS, _TOP_K), jnp.int32)],
        compiler_params=pltpu.CompilerParams(
            dimension_semantics=("arbitrary",),
        ),
    )(hidden_states, W_c, cid2d, lm_head_weight)
    return out


# speedup vs baseline: 1.0176x; 1.0176x over previous
import functools

import jax
import jax.numpy as jnp
import numpy as np
from jax import lax
from jax.experimental import pallas as pl
from jax.experimental.pallas import tpu as pltpu
from jax.experimental.pallas import tpu_sc as plsc

_HIDDEN = 1024
_VOCAB = 100000
_NUM_CENTROIDS = 1000
_TOP_K = 8
_PER_CENTROID = _VOCAB // _NUM_CENTROIDS
_NUM_TOKENS = 128
_FMIN = float(np.finfo(np.float32).min)

_SC_NC, _SC_NS = 2, 16
_SC_ACTIVE = 25
_SC_CHUNKS = _NUM_CENTROIDS // _SC_ACTIVE


_SC_PER_TILE = 3128
_VOCAB_PAD = _SC_PER_TILE * _SC_NC * _SC_NS
_SC_SLICE = _VOCAB_PAD // _SC_NS


def _sc_invperm(ord1d, vals1d, zeros1d):
    mesh = plsc.VectorSubcoreMesh(core_axis_name="c", subcore_axis_name="s")

    @functools.partial(
        pl.kernel,
        out_type=jax.ShapeDtypeStruct((_SC_NC * _VOCAB_PAD,), jnp.int32),
        mesh=mesh,
        scratch_types=[
            pltpu.VMEM((_SC_PER_TILE,), jnp.int32),
            pltpu.VMEM((_SC_PER_TILE,), jnp.int32),
            pltpu.VMEM((_SC_SLICE,), jnp.int32),
            pltpu.VMEM_SHARED((_VOCAB_PAD,), jnp.int32),
            pltpu.SemaphoreType.DMA,
        ],
    )
    def k(ord_hbm, vals_hbm, zeros_hbm, out_hbm, idx_v, val_v, zv, buf, sem):
        cid = lax.axis_index("c")
        sid = lax.axis_index("s")
        wid = sid * _SC_NC + cid
        pltpu.sync_copy(zeros_hbm, zv)
        pltpu.sync_copy(zv, buf.at[pl.ds(sid * _SC_SLICE, _SC_SLICE)])
        base = wid * _SC_PER_TILE
        pltpu.sync_copy(ord_hbm.at[pl.ds(base, _SC_PER_TILE)], idx_v)
        pltpu.sync_copy(vals_hbm.at[pl.ds(base, _SC_PER_TILE)], val_v)
        plsc.subcore_barrier()
        pltpu.async_copy(val_v, buf.at[idx_v], sem).wait()
        plsc.subcore_barrier()
        pltpu.sync_copy(buf.at[pl.ds(sid * _SC_SLICE, _SC_SLICE)], zv)
        pltpu.sync_copy(
            zv, out_hbm.at[pl.ds(cid * _VOCAB_PAD + sid * _SC_SLICE, _SC_SLICE)])

    return k(ord1d, vals1d, zeros1d)


def _topk_body(h_ref, wc_ref, top_ref):
    scores = lax.dot_general(
        h_ref[...], wc_ref[...], (((1,), (1,)), ((), ())),
        preferred_element_type=jnp.float32)
    iota = lax.broadcasted_iota(jnp.int32, scores.shape, 1)
    s = scores
    cols = []
    for _ in range(_TOP_K):
        m = jnp.max(s, axis=1, keepdims=True)
        idx = jnp.min(jnp.where(s >= m, iota, jnp.int32(2**30)),
                      axis=1, keepdims=True)
        cols.append(idx)
        s = jnp.where(iota == idx, _FMIN, s)
    top_ref[...] = jnp.concatenate(cols, axis=1)


_BLK_V = 5120


def _main_body(h_ref, top_ref, cid_ref, w_ref, out_ref):
    h = h_ref[...].astype(jnp.bfloat16)
    w = w_ref[...].astype(jnp.bfloat16)
    logits = lax.dot_general(
        h, w, (((1,), (1,)), ((), ())),
        preferred_element_type=jnp.float32)
    cid2 = cid_ref[...]
    cid = cid2[0:1] + cid2[1:2]
    top = top_ref[...]
    sel = cid == top[:, 0:1]
    for kk in range(1, _TOP_K):
        sel = sel | (cid == top[:, kk:kk + 1])
    out_ref[...] = jnp.where(sel, logits, jnp.float32(_FMIN))


def kernel(hidden_states, lm_head_weight, W_c, token_ordering):
    ord1d = jnp.concatenate([
        token_ordering,
        jnp.arange(_VOCAB, _VOCAB_PAD, dtype=jnp.int32)])
    vals1d = jnp.concatenate([
        jnp.arange(_VOCAB, dtype=jnp.int32) // _PER_CENTROID,
        jnp.zeros(_VOCAB_PAD - _VOCAB, jnp.int32)])
    zeros1d = jnp.zeros(_SC_SLICE, jnp.int32)

    cid2d = _sc_invperm(ord1d, vals1d, zeros1d).reshape(_SC_NC, _VOCAB_PAD)

    top8 = pl.pallas_call(
        _topk_body,
        out_shape=jax.ShapeDtypeStruct((_NUM_TOKENS, _TOP_K), jnp.int32),
    )(hidden_states, W_c)

    n_blocks = -(-_VOCAB // _BLK_V)
    out = pl.pallas_call(
        _main_body,
        grid=(n_blocks,),
        in_specs=[
            pl.BlockSpec((_NUM_TOKENS, _HIDDEN), lambda v: (0, 0)),
            pl.BlockSpec((_NUM_TOKENS, _TOP_K), lambda v: (0, 0)),
            pl.BlockSpec((_SC_NC, _BLK_V), lambda v: (0, v)),
            pl.BlockSpec((_BLK_V, _HIDDEN), lambda v: (v, 0)),
        ],
        out_specs=pl.BlockSpec((_NUM_TOKENS, _BLK_V), lambda v: (0, v)),
        out_shape=jax.ShapeDtypeStruct((_NUM_TOKENS, _VOCAB), jnp.float32),
        compiler_params=pltpu.CompilerParams(
            dimension_semantics=("arbitrary",),
        ),
    )(hidden_states, top8, cid2d, lm_head_weight)
    return out


# speedup vs baseline: 1.0200x; 1.0024x over previous
import functools

import jax
import jax.numpy as jnp
import numpy as np
from jax import lax
from jax.experimental import pallas as pl
from jax.experimental.pallas import tpu as pltpu
from jax.experimental.pallas import tpu_sc as plsc

_HIDDEN = 1024
_VOCAB = 100000
_NUM_CENTROIDS = 1000
_TOP_K = 8
_PER_CENTROID = _VOCAB // _NUM_CENTROIDS
_NUM_TOKENS = 128
_FMIN = float(np.finfo(np.float32).min)

_SC_NC, _SC_NS = 2, 16
_SC_ACTIVE = 25
_SC_CHUNKS = _NUM_CENTROIDS // _SC_ACTIVE


_SC_PER_TILE = 3128
_VOCAB_PAD = _SC_PER_TILE * _SC_NC * _SC_NS
_SC_SLICE = _VOCAB_PAD // _SC_NS


def _sc_invperm(ord1d, vals1d, zeros1d):
    mesh = plsc.VectorSubcoreMesh(core_axis_name="c", subcore_axis_name="s")

    @functools.partial(
        pl.kernel,
        out_type=jax.ShapeDtypeStruct((_SC_NC * _VOCAB_PAD,), jnp.int32),
        mesh=mesh,
        scratch_types=[
            pltpu.VMEM((_SC_PER_TILE,), jnp.int32),
            pltpu.VMEM((_SC_PER_TILE,), jnp.int32),
            pltpu.VMEM((_SC_SLICE,), jnp.int32),
            pltpu.VMEM_SHARED((_VOCAB_PAD,), jnp.int32),
            pltpu.SemaphoreType.DMA,
        ],
    )
    def k(ord_hbm, vals_hbm, zeros_hbm, out_hbm, idx_v, val_v, zv, buf, sem):
        cid = lax.axis_index("c")
        sid = lax.axis_index("s")
        wid = sid * _SC_NC + cid
        pltpu.sync_copy(zeros_hbm, zv)
        pltpu.sync_copy(zv, buf.at[pl.ds(sid * _SC_SLICE, _SC_SLICE)])
        base = wid * _SC_PER_TILE
        pltpu.sync_copy(ord_hbm.at[pl.ds(base, _SC_PER_TILE)], idx_v)
        pltpu.sync_copy(vals_hbm.at[pl.ds(base, _SC_PER_TILE)], val_v)
        plsc.subcore_barrier()
        pltpu.async_copy(val_v, buf.at[idx_v], sem).wait()
        plsc.subcore_barrier()
        pltpu.sync_copy(buf.at[pl.ds(sid * _SC_SLICE, _SC_SLICE)], zv)
        pltpu.sync_copy(
            zv, out_hbm.at[pl.ds(cid * _VOCAB_PAD + sid * _SC_SLICE, _SC_SLICE)])

    return k(ord1d, vals1d, zeros1d)


def _topk_body(h_ref, wc_ref, top_ref):
    scores = lax.dot_general(
        h_ref[...], wc_ref[...], (((1,), (1,)), ((), ())),
        preferred_element_type=jnp.float32)
    iota = lax.broadcasted_iota(jnp.int32, scores.shape, 1)
    s = scores
    cols = []
    for _ in range(_TOP_K):
        m = jnp.max(s, axis=1, keepdims=True)
        idx = jnp.min(jnp.where(s >= m, iota, jnp.int32(2**30)),
                      axis=1, keepdims=True)
        cols.append(idx)
        s = jnp.where(iota == idx, _FMIN, s)
    top_ref[...] = jnp.concatenate(cols, axis=1)


_BLK_V = 5120


def _main_body(h_ref, top_ref, cid_ref, wa_ref, wb_ref, out_ref):
    h = h_ref[...].astype(jnp.bfloat16)
    cid2 = cid_ref[...]
    cid = cid2[0:1] + cid2[1:2]
    top = top_ref[...]
    sel = cid == top[:, 0:1]
    for kk in range(1, _TOP_K):
        sel = sel | (cid == top[:, kk:kk + 1])
    half = _BLK_V // 2
    for i, w_ref in enumerate((wa_ref, wb_ref)):
        w = w_ref[...].astype(jnp.bfloat16)
        logits = lax.dot_general(
            h, w, (((1,), (1,)), ((), ())),
            preferred_element_type=jnp.float32)
        out_ref[:, i * half:(i + 1) * half] = jnp.where(
            sel[:, i * half:(i + 1) * half], logits, jnp.float32(_FMIN))


def kernel(hidden_states, lm_head_weight, W_c, token_ordering):
    ord1d = jnp.concatenate([
        token_ordering,
        jnp.arange(_VOCAB, _VOCAB_PAD, dtype=jnp.int32)])
    vals1d = jnp.concatenate([
        jnp.arange(_VOCAB, dtype=jnp.int32) // _PER_CENTROID,
        jnp.zeros(_VOCAB_PAD - _VOCAB, jnp.int32)])
    zeros1d = jnp.zeros(_SC_SLICE, jnp.int32)

    cid2d = _sc_invperm(ord1d, vals1d, zeros1d).reshape(_SC_NC, _VOCAB_PAD)

    top8 = pl.pallas_call(
        _topk_body,
        out_shape=jax.ShapeDtypeStruct((_NUM_TOKENS, _TOP_K), jnp.int32),
    )(hidden_states, W_c)

    n_blocks = -(-_VOCAB // _BLK_V)
    out = pl.pallas_call(
        _main_body,
        grid=(n_blocks,),
        in_specs=[
            pl.BlockSpec((_NUM_TOKENS, _HIDDEN), lambda v: (0, 0)),
            pl.BlockSpec((_NUM_TOKENS, _TOP_K), lambda v: (0, 0)),
            pl.BlockSpec((_SC_NC, _BLK_V), lambda v: (0, v)),
            pl.BlockSpec((_BLK_V // 2, _HIDDEN), lambda v: (2 * v, 0)),
            pl.BlockSpec((_BLK_V // 2, _HIDDEN), lambda v: (2 * v + 1, 0)),
        ],
        out_specs=pl.BlockSpec((_NUM_TOKENS, _BLK_V), lambda v: (0, v)),
        out_shape=jax.ShapeDtypeStruct((_NUM_TOKENS, _VOCAB), jnp.float32),
        compiler_params=pltpu.CompilerParams(
            dimension_semantics=("arbitrary",),
        ),
    )(hidden_states, top8, cid2d, lm_head_weight, lm_head_weight)
    return out
